# baseline (speedup 1.0000x reference)
import functools

import jax
import jax.numpy as jnp
from jax import lax
from jax.experimental import pallas as pl
from jax.experimental.pallas import tpu as pltpu
from jax.experimental.pallas import tpu_sc as plsc

_LOG_2PI = 1.8378770664093453


def _sc_gather(times, mean_t, std_t, b, d, n_times):
    info = plsc.get_sparse_core_info()
    nw = info.num_cores * info.num_subcores
    fpw = d // nw
    q = 4096
    nq = b // q

    mesh = plsc.VectorSubcoreMesh(core_axis_name="c", subcore_axis_name="s")

    @functools.partial(
        pl.kernel,
        mesh=mesh,
        out_type=[
            jax.ShapeDtypeStruct((d, b), jnp.float32),
            jax.ShapeDtypeStruct((d, b), jnp.float32),
        ],
        scratch_types=[
            pltpu.VMEM((n_times,), jnp.float32),
            pltpu.VMEM((b,), jnp.int32),
            pltpu.VMEM((2, q), jnp.float32),
            pltpu.SemaphoreType.DMA,
            pltpu.SemaphoreType.DMA,
            pltpu.SemaphoreType.DMA,
        ],
        compiler_params=pltpu.CompilerParams(
            use_tc_tiling_on_sc=True,
            needs_layout_passes=False,
        ),
    )
    def run(times_hbm, mean_hbm, std_hbm, mu_out, sd_out,
            col_v, idx_v, st_v, csem, wsem0, wsem1):
        wid = lax.axis_index("s") * info.num_cores + lax.axis_index("c")
        passes = [(mean_hbm, mu_out, 0), (std_hbm, sd_out, 0),
                  (mean_hbm, mu_out, 1), (std_hbm, sd_out, 1)]
        wsems = [wsem0, wsem1]
        wr = [None, None]

        cp = pltpu.async_copy(mean_hbm.at[wid * fpw], col_v, csem)
        pltpu.sync_copy(times_hbm, idx_v)
        for p, (tbl, out, fi) in enumerate(passes):
            f = wid * fpw + fi
            cp.wait()
            for k in range(nq):
                sl = k & 1
                if wr[sl] is not None:
                    wr[sl].wait()

                @plsc.parallel_loop(0, q, 16, unroll=16)
                def _(i):
                    iv = idx_v[pl.ds(k * q + i, 16)]
                    st_v[sl, pl.ds(i, 16)] = plsc.load_gather(col_v, [iv])

                if k == nq - 1 and p + 1 < len(passes):
                    tbl2, _out2, fi2 = passes[p + 1]
                    cp = pltpu.async_copy(
                        tbl2.at[wid * fpw + fi2], col_v, csem)
                wr[sl] = pltpu.async_copy(
                    st_v.at[sl], out.at[f, pl.ds(k * q, q)], wsems[sl])
        wr[0].wait()
        wr[1].wait()

    return run(times, mean_t, std_t)


def _tc_log_prob(states_t, mu_g, sd_g, b, d):
    blk = 8192
    grid = b // blk

    def body(st_ref, mu_ref, sd_ref, out_ref):
        s = st_ref[...]
        m = mu_ref[...]
        sig = jnp.maximum(sd_ref[...], 0.01)
        t = (s - m) / sig
        lp = -0.5 * (t * t) - jnp.log(sig)
        out_ref[0, 0, :] = jnp.sum(lp, axis=0) - (d * 0.5) * _LOG_2PI

    out = pl.pallas_call(
        body,
        grid=(grid,),
        in_specs=[
            pl.BlockSpec((d, blk), lambda i: (0, i)),
            pl.BlockSpec((d, blk), lambda i: (0, i)),
            pl.BlockSpec((d, blk), lambda i: (0, i)),
        ],
        out_specs=pl.BlockSpec((1, 1, blk), lambda i: (i, 0, 0)),
        out_shape=jax.ShapeDtypeStruct((grid, 1, blk), jnp.float32),
    )(states_t, mu_g, sd_g)
    return out.reshape(b)


def kernel(states, times, mean_params, std_params):
    b, d = states.shape
    n_times = mean_params.shape[0]
    times = times.reshape(-1).astype(jnp.int32)
    mu_g, sd_g = _sc_gather(times, mean_params.T, std_params.T, b, d, n_times)
    return _tc_log_prob(states.T, mu_g, sd_g, b, d)


# speedup vs baseline: 1.0005x; 1.0005x over previous
import functools

import jax
import jax.numpy as jnp
from jax import lax
from jax.experimental import pallas as pl
from jax.experimental.pallas import tpu as pltpu
from jax.experimental.pallas import tpu_sc as plsc

_LOG_2PI = 1.8378770664093453


def _sc_gather(times, mean_t, std_t, b, d, n_times):
    info = plsc.get_sparse_core_info()
    nw = info.num_cores * info.num_subcores
    fpw = d // nw
    q = 4096
    nq = b // q

    mesh = plsc.VectorSubcoreMesh(core_axis_name="c", subcore_axis_name="s")

    @functools.partial(
        pl.kernel,
        mesh=mesh,
        out_type=[
            jax.ShapeDtypeStruct((d, b), jnp.float32),
            jax.ShapeDtypeStruct((d, b), jnp.float32),
        ],
        scratch_types=[
            pltpu.VMEM((n_times,), jnp.float32),
            pltpu.VMEM((b,), jnp.int32),
            pltpu.VMEM((2, q), jnp.float32),
            pltpu.SemaphoreType.DMA,
            pltpu.SemaphoreType.DMA,
            pltpu.SemaphoreType.DMA,
        ],
        compiler_params=pltpu.CompilerParams(
            use_tc_tiling_on_sc=True,
            needs_layout_passes=False,
        ),
    )
    def run(times_hbm, mean_hbm, std_hbm, mu_out, sd_out,
            col_v, idx_v, st_v, csem, wsem0, wsem1):
        wid = lax.axis_index("s") * info.num_cores + lax.axis_index("c")
        passes = [(mean_hbm, mu_out, 0), (std_hbm, sd_out, 0),
                  (mean_hbm, mu_out, 1), (std_hbm, sd_out, 1)]
        wsems = [wsem0, wsem1]
        wr = [None, None]

        cp = pltpu.async_copy(mean_hbm.at[wid * fpw], col_v, csem)
        pltpu.sync_copy(times_hbm, idx_v)
        for p, (tbl, out, fi) in enumerate(passes):
            f = wid * fpw + fi
            cp.wait()
            for k in range(nq):
                sl = k & 1
                if wr[sl] is not None:
                    wr[sl].wait()

                @plsc.parallel_loop(0, q, 16, unroll=4)
                def _(i):
                    iv = idx_v[pl.ds(k * q + i, 16)]
                    st_v[sl, pl.ds(i, 16)] = plsc.load_gather(col_v, [iv])

                if k == nq - 1 and p + 1 < len(passes):
                    tbl2, _out2, fi2 = passes[p + 1]
                    cp = pltpu.async_copy(
                        tbl2.at[wid * fpw + fi2], col_v, csem)
                wr[sl] = pltpu.async_copy(
                    st_v.at[sl], out.at[f, pl.ds(k * q, q)], wsems[sl])
        wr[0].wait()
        wr[1].wait()

    return run(times, mean_t, std_t)


def _tc_log_prob(states_t, mu_g, sd_g, b, d):
    blk = 4096
    grid = b // blk

    def body(st_ref, mu_ref, sd_ref, out_ref):
        s = st_ref[...]
        m = mu_ref[...]
        sig = jnp.maximum(sd_ref[...], 0.01)
        t = (s - m) / sig
        lp = -0.5 * (t * t) - jnp.log(sig)
        out_ref[0, 0, :] = jnp.sum(lp, axis=0) - (d * 0.5) * _LOG_2PI

    out = pl.pallas_call(
        body,
        grid=(grid,),
        in_specs=[
            pl.BlockSpec((d, blk), lambda i: (0, i)),
            pl.BlockSpec((d, blk), lambda i: (0, i)),
            pl.BlockSpec((d, blk), lambda i: (0, i)),
        ],
        out_specs=pl.BlockSpec((1, 1, blk), lambda i: (i, 0, 0)),
        out_shape=jax.ShapeDtypeStruct((grid, 1, blk), jnp.float32),
    )(states_t, mu_g, sd_g)
    return out.reshape(b)


def kernel(states, times, mean_params, std_params):
    b, d = states.shape
    n_times = mean_params.shape[0]
    times = times.reshape(-1).astype(jnp.int32)
    mu_g, sd_g = _sc_gather(times, mean_params.T, std_params.T, b, d, n_times)
    return _tc_log_prob(states.T, mu_g, sd_g, b, d)


# speedup vs baseline: 1.0079x; 1.0073x over previous
import functools

import jax
import jax.numpy as jnp
from jax import lax
from jax.experimental import pallas as pl
from jax.experimental.pallas import tpu as pltpu
from jax.experimental.pallas import tpu_sc as plsc

_LOG_2PI = 1.8378770664093453


def _sc_gather(times, mean_t, std_t, b, d, n_times):
    info = plsc.get_sparse_core_info()
    nw = info.num_cores * info.num_subcores
    fpw = d // nw
    q = 4096
    nq = b // q

    mesh = plsc.VectorSubcoreMesh(core_axis_name="c", subcore_axis_name="s")

    @functools.partial(
        pl.kernel,
        mesh=mesh,
        out_type=[
            jax.ShapeDtypeStruct((d, b), jnp.float32),
            jax.ShapeDtypeStruct((d, b), jnp.float32),
        ],
        scratch_types=[
            pltpu.VMEM((n_times,), jnp.float32),
            pltpu.VMEM((b,), jnp.int32),
            pltpu.VMEM((2, q), jnp.float32),
            pltpu.SemaphoreType.DMA,
            pltpu.SemaphoreType.DMA,
            pltpu.SemaphoreType.DMA,
        ],
        compiler_params=pltpu.CompilerParams(
            use_tc_tiling_on_sc=True,
            needs_layout_passes=False,
        ),
    )
    def run(times_hbm, mean_hbm, std_hbm, mu_out, sd_out,
            col_v, idx_v, st_v, csem, wsem0, wsem1):
        wid = lax.axis_index("s") * info.num_cores + lax.axis_index("c")
        passes = [(mean_hbm, mu_out, 0), (std_hbm, sd_out, 0),
                  (mean_hbm, mu_out, 1), (std_hbm, sd_out, 1)]
        wsems = [wsem0, wsem1]
        wr = [None, None]

        cp = pltpu.async_copy(mean_hbm.at[wid * fpw], col_v, csem)
        pltpu.sync_copy(times_hbm, idx_v)
        for p, (tbl, out, fi) in enumerate(passes):
            f = wid * fpw + fi
            cp.wait()
            for k in range(nq):
                sl = k & 1
                if wr[sl] is not None:
                    wr[sl].wait()

                @plsc.parallel_loop(0, q, 16, unroll=8)
                def _(i):
                    iv = idx_v[pl.ds(k * q + i, 16)]
                    st_v[sl, pl.ds(i, 16)] = plsc.load_gather(col_v, [iv])

                if k == nq - 1 and p + 1 < len(passes):
                    tbl2, _out2, fi2 = passes[p + 1]
                    cp = pltpu.async_copy(
                        tbl2.at[wid * fpw + fi2], col_v, csem)
                wr[sl] = pltpu.async_copy(
                    st_v.at[sl], out.at[f, pl.ds(k * q, q)], wsems[sl])
        wr[0].wait()
        wr[1].wait()

    return run(times, mean_t, std_t)


def _tc_log_prob(states_t, mu_g, sd_g, b, d):
    blk = 4096
    grid = b // blk

    def body(st_ref, mu_ref, sd_ref, out_ref):
        s = st_ref[...]
        m = mu_ref[...]
        sig = jnp.maximum(sd_ref[...], 0.01)
        t = (s - m) / sig
        lp = -0.5 * (t * t) - jnp.log(sig)
        out_ref[0, 0, :] = jnp.sum(lp, axis=0) - (d * 0.5) * _LOG_2PI

    out = pl.pallas_call(
        body,
        grid=(grid,),
        in_specs=[
            pl.BlockSpec((d, blk), lambda i: (0, i)),
            pl.BlockSpec((d, blk), lambda i: (0, i)),
            pl.BlockSpec((d, blk), lambda i: (0, i)),
        ],
        out_specs=pl.BlockSpec((1, 1, blk), lambda i: (i, 0, 0)),
        out_shape=jax.ShapeDtypeStruct((grid, 1, blk), jnp.float32),
    )(states_t, mu_g, sd_g)
    return out.reshape(b)


def kernel(states, times, mean_params, std_params):
    b, d = states.shape
    n_times = mean_params.shape[0]
    times = times.reshape(-1).astype(jnp.int32)
    mu_g, sd_g = _sc_gather(times, mean_params.T, std_params.T, b, d, n_times)
    return _tc_log_prob(states.T, mu_g, sd_g, b, d)
